# Initial kernel scaffold; baseline (speedup 1.0000x reference)
import jax
import jax.numpy as jnp
from jax.experimental import pallas as pl


def kernel(x, weight):
    raise NotImplementedError("write your pallas kernel here")


# speedup vs baseline: 1.8461x; 1.8461x over previous
import functools

import jax
import jax.numpy as jnp
from jax import lax
from jax.experimental import pallas as pl
from jax.experimental.pallas import tpu as pltpu
from jax.experimental.pallas import tpu_sc as plsc

B0, B1 = 16384, 50
D = 64
TOKENS = B0 * B1
NW = 32
IDXW = 128
NROWS = TOKENS // IDXW
ROWS_PER_W = NROWS // NW
CHUNK_IR = 8
NCHUNK = ROWS_PER_W // CHUNK_IR

_mesh = plsc.VectorSubcoreMesh(core_axis_name="c", subcore_axis_name="s")


@functools.partial(
    pl.kernel,
    mesh=_mesh,
    compiler_params=pltpu.CompilerParams(use_tc_tiling_on_sc=False),
    out_type=jax.ShapeDtypeStruct((NROWS, IDXW, D), jnp.float32),
    scratch_types=[
        pltpu.VMEM((CHUNK_IR, IDXW), jnp.int32),
        pltpu.VMEM((CHUNK_IR, IDXW, D), jnp.float32),
        pltpu.SemaphoreType.DMA,
    ],
)
def _gather(idx_hbm, table_hbm, out_hbm, idx_v, rows_v, sem):
    wid = lax.axis_index("s") * 2 + lax.axis_index("c")
    row0 = wid * ROWS_PER_W

    def chunk_body(c, carry):
        r = row0 + c * CHUNK_IR
        pltpu.sync_copy(idx_hbm.at[pl.ds(r, CHUNK_IR)], idx_v)
        handles = [
            pltpu.async_copy(table_hbm.at[idx_v.at[j]], rows_v.at[j], sem)
            for j in range(CHUNK_IR)
        ]
        for h in handles:
            h.wait()
        pltpu.sync_copy(rows_v, out_hbm.at[pl.ds(r, CHUNK_IR)])
        return carry

    lax.fori_loop(0, NCHUNK, chunk_body, 0)


def kernel(x, weight):
    idx = x.astype(jnp.int32).reshape(NROWS, IDXW)
    out = _gather(idx, weight)
    return out.reshape(B0, B1, D)


# speedup vs baseline: 1.8736x; 1.0149x over previous
import functools

import jax
import jax.numpy as jnp
from jax import lax
from jax.experimental import pallas as pl
from jax.experimental.pallas import tpu as pltpu
from jax.experimental.pallas import tpu_sc as plsc

B0, B1 = 16384, 50
D = 64
TOKENS = B0 * B1
NW = 32
IDXW = 128
NROWS = TOKENS // IDXW
ROWS_PER_W = NROWS // NW
CHUNK_IR = 5
NCHUNK = ROWS_PER_W // CHUNK_IR
NBUF = 2
NROUNDS = NCHUNK // NBUF

_mesh = plsc.VectorSubcoreMesh(core_axis_name="c", subcore_axis_name="s")


@functools.partial(
    pl.kernel,
    mesh=_mesh,
    compiler_params=pltpu.CompilerParams(use_tc_tiling_on_sc=False),
    out_type=jax.ShapeDtypeStruct((NROWS, IDXW, D), jnp.float32),
    scratch_types=[
        pltpu.VMEM((ROWS_PER_W, IDXW), jnp.int32),
        pltpu.VMEM((NBUF, CHUNK_IR, IDXW, D), jnp.float32),
        pltpu.SemaphoreType.DMA,
        pltpu.SemaphoreType.DMA,
        pltpu.SemaphoreType.DMA,
        pltpu.SemaphoreType.DMA,
    ],
)
def _gather(idx_hbm, table_hbm, out_hbm, idx_all, rows, g0, g1, o0, o1):
    gsem = (g0, g1)
    osem = (o0, o1)
    wid = lax.axis_index("s") * 2 + lax.axis_index("c")
    row0 = wid * ROWS_PER_W

    pltpu.sync_copy(idx_hbm.at[pl.ds(row0, ROWS_PER_W)], idx_all)

    def fire(c, b):
        for j in range(CHUNK_IR):
            pltpu.async_copy(table_hbm.at[idx_all.at[c * CHUNK_IR + j]],
                             rows.at[b, j], gsem[b])

    def drain_gather(c, b):
        for j in range(CHUNK_IR):
            pltpu.make_async_copy(table_hbm.at[idx_all.at[c * CHUNK_IR + j]],
                                  rows.at[b, j], gsem[b]).wait()

    def start_out(c, b):
        pltpu.async_copy(rows.at[b],
                         out_hbm.at[pl.ds(row0 + c * CHUNK_IR, CHUNK_IR)],
                         osem[b])

    def drain_out(c, b):
        pltpu.make_async_copy(rows.at[b],
                              out_hbm.at[pl.ds(row0 + c * CHUNK_IR, CHUNK_IR)],
                              osem[b]).wait()

    fire(0, 0)
    fire(1, 1)
    drain_gather(0, 0)
    start_out(0, 0)

    def round_body(r, carry):
        for b in range(NBUF):
            c = r * NBUF + b
            drain_out(c - NBUF, b)
            fire(c, b)
            pb = 1 - b
            drain_gather(c - 1, pb)
            start_out(c - 1, pb)
        return carry

    lax.fori_loop(1, NROUNDS, round_body, 0)

    last = NCHUNK - 1
    drain_gather(last, 1)
    start_out(last, 1)
    drain_out(last - 1, 0)
    drain_out(last, 1)


def kernel(x, weight):
    idx = x.astype(jnp.int32).reshape(NROWS, IDXW)
    out = _gather(idx, weight)
    return out.reshape(B0, B1, D)
